# baseline (speedup 1.0000x reference)
import functools

import jax
import jax.numpy as jnp
from jax import lax
from jax.experimental import pallas as pl
from jax.experimental.pallas import tpu as pltpu
from jax.experimental.pallas import tpu_sc as plsc

_K = 1024
_D = 64
_B = 16
_HW = 1024
_N = _B * _HW


def _tc_body(lat_ref, e_ref, inds_ref, loss_ref):
    b = pl.program_id(0)
    lat = lat_ref[0]
    emb = e_ref[...]
    scores = lax.dot_general(
        emb, lat, (((1,), (0,)), ((), ())),
        preferred_element_type=jnp.float32)
    enorm = jnp.sum(emb * emb, axis=1, keepdims=True)
    fnorm = jnp.sum(lat * lat, axis=0, keepdims=True)
    dist = (fnorm + enorm) - 2.0 * scores
    inds_ref[0, 0, :] = jnp.argmin(dist, axis=0).astype(jnp.int32)
    s = jnp.sum(jnp.min(dist, axis=0, keepdims=True), axis=1, keepdims=True)

    @pl.when(b == 0)
    def _init():
        loss_ref[...] = jnp.zeros_like(s)

    loss_ref[...] += s


def _tc_argmin(lat3, emb):
    return pl.pallas_call(
        _tc_body,
        grid=(_B,),
        in_specs=[
            pl.BlockSpec((1, _D, _HW), lambda b: (b, 0, 0)),
            pl.BlockSpec((_K, _D), lambda b: (0, 0)),
        ],
        out_specs=[
            pl.BlockSpec((1, 1, _HW), lambda b: (b, 0, 0)),
            pl.BlockSpec((1, 1), lambda b: (0, 0)),
        ],
        out_shape=[
            jax.ShapeDtypeStruct((_B, 1, _HW), jnp.int32),
            jax.ShapeDtypeStruct((1, 1), jnp.float32),
        ],
    )(lat3, emb)


_NC = 2
_NS = 16
_NW = _NC * _NS
_BPW = _N // _NW

@functools.cache
def _sc_gather_fn():
    mesh = plsc.VectorSubcoreMesh(
        core_axis_name="c", subcore_axis_name="s",
        num_cores=_NC, num_subcores=_NS)

    @functools.partial(
        pl.kernel, mesh=mesh,
        compiler_params=pltpu.CompilerParams(needs_layout_passes=False),
        out_type=jax.ShapeDtypeStruct((_B * _D, 8, 128), jnp.float32),
        scratch_types=[
            pltpu.VMEM((_K // 2, 128), jnp.float32),
            pltpu.VMEM((_BPW,), jnp.int32),
            pltpu.VMEM((_D, _BPW // 128, 128), jnp.float32),
        ],
    )
    def _sc_gather(table_hbm, idx_hbm, out_hbm, e_v, idx_v, out_v):
        wid = lax.axis_index("s") * _NC + lax.axis_index("c")
        b = wid // 2
        half = wid % 2
        pltpu.sync_copy(table_hbm, e_v)
        pltpu.sync_copy(idx_hbm.at[pl.ds(wid * _BPW, _BPW)], idx_v)

        def jbody(j, carry):
            idx16 = idx_v[pl.ds(j * 16, 16)]
            rows = lax.shift_right_logical(idx16, 1)
            colbase = lax.shift_left(jnp.bitwise_and(idx16, 1), 6)
            q = j // 8
            r = (j % 8) * 16
            for c in range(_D):
                out_v[c, q, pl.ds(r, 16)] = plsc.load_gather(
                    e_v, [rows, colbase + c])
            return carry

        lax.fori_loop(0, _BPW // 16, jbody, 0)
        pltpu.sync_copy(
            out_v, out_hbm.at[pl.ds(b * _D, _D), pl.ds(half * 4, 4)])

    return _sc_gather


def kernel(latents, embedding_weight):
    lat3 = latents.reshape(_B, _D, _HW)
    inds3, losssum = _tc_argmin(lat3, embedding_weight)
    inds = inds3.reshape(_N)
    q = _sc_gather_fn()(embedding_weight.reshape(_K // 2, 128), inds)
    out = q.reshape(_B, _D, 32, 32)
    vq_loss = losssum[0, 0] * (1.25 / _N / _D)
    return out, vq_loss


# speedup vs baseline: 1.5392x; 1.5392x over previous
import functools

import jax
import jax.numpy as jnp
from jax import lax
from jax.experimental import pallas as pl
from jax.experimental.pallas import tpu as pltpu
from jax.experimental.pallas import tpu_sc as plsc

_K = 1024
_D = 64
_B = 16
_HW = 1024
_N = _B * _HW


def _tc_body(et_ref, flat_ref, fnorm_ref, enorm_ref, inds_ref, loss_ref):
    b = pl.program_id(0)
    et = et_ref[...]
    fb = flat_ref[0]
    scores = lax.dot_general(
        et, fb, (((0,), (1,)), ((), ())),
        preferred_element_type=jnp.float32)
    dist = (fnorm_ref[0] + enorm_ref[...]) - 2.0 * scores
    inds_ref[0, 0, :] = jnp.argmin(dist, axis=0).astype(jnp.int32)
    s = jnp.sum(jnp.min(dist, axis=0, keepdims=True), axis=1, keepdims=True)

    @pl.when(b == 0)
    def _init():
        loss_ref[...] = jnp.zeros_like(s)

    loss_ref[...] += s


def _tc_argmin(et, flat3, fnorm3, enorm2):
    return pl.pallas_call(
        _tc_body,
        grid=(_B,),
        in_specs=[
            pl.BlockSpec((_D, _K), lambda b: (0, 0)),
            pl.BlockSpec((1, _HW, _D), lambda b: (b, 0, 0)),
            pl.BlockSpec((1, 1, _HW), lambda b: (b, 0, 0)),
            pl.BlockSpec((_K, 1), lambda b: (0, 0)),
        ],
        out_specs=[
            pl.BlockSpec((1, 1, _HW), lambda b: (b, 0, 0)),
            pl.BlockSpec((1, 1), lambda b: (0, 0)),
        ],
        out_shape=[
            jax.ShapeDtypeStruct((_B, 1, _HW), jnp.int32),
            jax.ShapeDtypeStruct((1, 1), jnp.float32),
        ],
    )(et, flat3, fnorm3, enorm2)


_NC = 2
_NS = 16
_NW = _NC * _NS
_BPW = _N // _NW


@functools.cache
def _sc_gather_fn():
    mesh = plsc.VectorSubcoreMesh(
        core_axis_name="c", subcore_axis_name="s",
        num_cores=_NC, num_subcores=_NS)

    @functools.partial(
        pl.kernel, mesh=mesh,
        out_type=jax.ShapeDtypeStruct((_N, 128), jnp.float32),
        scratch_types=[
            pltpu.VMEM((_BPW,), jnp.int32),
            pltpu.VMEM((_BPW, 128), jnp.float32),
            pltpu.SemaphoreType.DMA,
        ],
    )
    def _sc_gather(table_hbm, idx_hbm, out_hbm, idx_v, rows_v, sem):
        wid = lax.axis_index("s") * _NC + lax.axis_index("c")
        base = wid * _BPW
        pltpu.sync_copy(idx_hbm.at[pl.ds(base, _BPW)], idx_v)
        pltpu.async_copy(table_hbm.at[idx_v], rows_v, sem).wait()
        pltpu.sync_copy(rows_v, out_hbm.at[pl.ds(base, _BPW)])

    return _sc_gather


def kernel(latents, embedding_weight):
    flat = jnp.transpose(latents, (0, 2, 3, 1)).reshape(_N, _D)
    et = embedding_weight.T
    fnorm3 = jnp.sum(flat ** 2, axis=1).reshape(_B, 1, _HW)
    enorm2 = jnp.sum(embedding_weight ** 2, axis=1).reshape(_K, 1)
    inds3, losssum = _tc_argmin(et, flat.reshape(_B, _HW, _D),
                                fnorm3, enorm2)
    inds = inds3.reshape(_N)
    table = jnp.pad(embedding_weight, ((0, 0), (0, 128 - _D)))
    q = _sc_gather_fn()(table, inds)
    out = q[:, :_D].reshape(_B, 32, 32, _D).transpose(0, 3, 1, 2)
    vq_loss = losssum[0, 0] * (1.25 / _N / _D)
    return out, vq_loss


# speedup vs baseline: 1.5907x; 1.0335x over previous
import functools

import jax
import jax.numpy as jnp
from jax import lax
from jax.experimental import pallas as pl
from jax.experimental.pallas import tpu as pltpu
from jax.experimental.pallas import tpu_sc as plsc

_K = 1024
_D = 64
_B = 16
_HW = 1024
_N = _B * _HW


def _tc_body(et_ref, flat_ref, enorm_ref, inds_ref, loss_ref):
    b = pl.program_id(0)
    et = et_ref[...]
    fb = flat_ref[0]
    scores = lax.dot_general(
        et, fb, (((0,), (1,)), ((), ())),
        preferred_element_type=jnp.float32)
    fnorm = jnp.sum(fb * fb, axis=1, keepdims=True)
    fnorm_row = lax.transpose(fnorm, (1, 0))
    dist = (fnorm_row + enorm_ref[...]) - 2.0 * scores
    inds_ref[0, 0, :] = jnp.argmin(dist, axis=0).astype(jnp.int32)
    s = jnp.sum(jnp.min(dist, axis=0, keepdims=True), axis=1, keepdims=True)

    @pl.when(b == 0)
    def _init():
        loss_ref[...] = jnp.zeros_like(s)

    loss_ref[...] += s


def _tc_argmin(et, flat3, enorm2):
    return pl.pallas_call(
        _tc_body,
        grid=(_B,),
        in_specs=[
            pl.BlockSpec((_D, _K), lambda b: (0, 0)),
            pl.BlockSpec((1, _HW, _D), lambda b: (b, 0, 0)),
            pl.BlockSpec((_K, 1), lambda b: (0, 0)),
        ],
        out_specs=[
            pl.BlockSpec((1, 1, _HW), lambda b: (b, 0, 0)),
            pl.BlockSpec((1, 1), lambda b: (0, 0)),
        ],
        out_shape=[
            jax.ShapeDtypeStruct((_B, 1, _HW), jnp.int32),
            jax.ShapeDtypeStruct((1, 1), jnp.float32),
        ],
    )(et, flat3, enorm2)


_NC = 2
_NS = 16
_NW = _NC * _NS
_BPW = _N // _NW


@functools.cache
def _sc_gather_fn():
    mesh = plsc.VectorSubcoreMesh(
        core_axis_name="c", subcore_axis_name="s",
        num_cores=_NC, num_subcores=_NS)

    @functools.partial(
        pl.kernel, mesh=mesh,
        out_type=jax.ShapeDtypeStruct((_N, 128), jnp.float32),
        scratch_types=[
            pltpu.VMEM((_BPW,), jnp.int32),
            pltpu.VMEM((_BPW, 128), jnp.float32),
            pltpu.SemaphoreType.DMA,
        ],
    )
    def _sc_gather(table_hbm, idx_hbm, out_hbm, idx_v, rows_v, sem):
        wid = lax.axis_index("s") * _NC + lax.axis_index("c")
        base = wid * _BPW
        pltpu.sync_copy(idx_hbm.at[pl.ds(base, _BPW)], idx_v)
        pltpu.async_copy(table_hbm.at[idx_v], rows_v, sem).wait()
        pltpu.sync_copy(rows_v, out_hbm.at[pl.ds(base, _BPW)])

    return _sc_gather


def kernel(latents, embedding_weight):
    flat = jnp.transpose(latents, (0, 2, 3, 1)).reshape(_N, _D)
    et = embedding_weight.T
    enorm2 = jnp.sum(embedding_weight ** 2, axis=1).reshape(_K, 1)
    inds3, losssum = _tc_argmin(et, flat.reshape(_B, _HW, _D), enorm2)
    inds = inds3.reshape(_N)
    table = jnp.pad(embedding_weight, ((0, 0), (0, 128 - _D)))
    q = _sc_gather_fn()(table, inds)
    out = q[:, :_D].reshape(_B, 32, 32, _D).transpose(0, 3, 1, 2)
    vq_loss = losssum[0, 0] * (1.25 / _N / _D)
    return out, vq_loss


# speedup vs baseline: 1.6599x; 1.0435x over previous
import functools

import jax
import jax.numpy as jnp
from jax import lax
from jax.experimental import pallas as pl
from jax.experimental.pallas import tpu as pltpu
from jax.experimental.pallas import tpu_sc as plsc

_K = 1024
_D = 64
_B = 16
_HW = 1024
_N = _B * _HW


def _tc_body(lat_ref, e_ref, inds_ref, loss_ref):
    b = pl.program_id(0)
    lat = lat_ref[0]
    emb = e_ref[...]
    scores = lax.dot_general(
        emb, lat, (((1,), (0,)), ((), ())),
        preferred_element_type=jnp.float32)
    enorm = jnp.sum(emb * emb, axis=1, keepdims=True)
    fnorm = jnp.sum(lat * lat, axis=0, keepdims=True)
    dist = (fnorm + enorm) - 2.0 * scores
    inds_ref[0, 0, :] = jnp.argmin(dist, axis=0).astype(jnp.int32)
    s = jnp.sum(jnp.min(dist, axis=0, keepdims=True), axis=1, keepdims=True)

    @pl.when(b == 0)
    def _init():
        loss_ref[...] = jnp.zeros_like(s)

    loss_ref[...] += s


def _tc_argmin(lat3, emb):
    return pl.pallas_call(
        _tc_body,
        grid=(_B,),
        in_specs=[
            pl.BlockSpec((1, _D, _HW), lambda b: (b, 0, 0)),
            pl.BlockSpec((_K, _D), lambda b: (0, 0)),
        ],
        out_specs=[
            pl.BlockSpec((1, 1, _HW), lambda b: (b, 0, 0)),
            pl.BlockSpec((1, 1), lambda b: (0, 0)),
        ],
        out_shape=[
            jax.ShapeDtypeStruct((_B, 1, _HW), jnp.int32),
            jax.ShapeDtypeStruct((1, 1), jnp.float32),
        ],
    )(lat3, emb)


_NC = 2
_NS = 16
_NW = _NC * _NS
_BPW = _N // _NW


@functools.cache
def _sc_gather_fn():
    mesh = plsc.VectorSubcoreMesh(
        core_axis_name="c", subcore_axis_name="s",
        num_cores=_NC, num_subcores=_NS)

    @functools.partial(
        pl.kernel, mesh=mesh,
        out_type=jax.ShapeDtypeStruct((_N, 128), jnp.float32),
        scratch_types=[
            pltpu.VMEM((_BPW,), jnp.int32),
            pltpu.VMEM((_BPW, 128), jnp.float32),
            pltpu.SemaphoreType.DMA,
        ],
    )
    def _sc_gather(table_hbm, idx_hbm, out_hbm, idx_v, rows_v, sem):
        wid = lax.axis_index("s") * _NC + lax.axis_index("c")
        base = wid * _BPW
        pltpu.sync_copy(idx_hbm.at[pl.ds(base, _BPW)], idx_v)
        pltpu.async_copy(table_hbm.at[idx_v], rows_v, sem).wait()
        pltpu.sync_copy(rows_v, out_hbm.at[pl.ds(base, _BPW)])

    return _sc_gather


def kernel(latents, embedding_weight):
    lat3 = latents.reshape(_B, _D, _HW)
    inds3, losssum = _tc_argmin(lat3, embedding_weight)
    inds = inds3.reshape(_N)
    table = jnp.pad(embedding_weight, ((0, 0), (0, 128 - _D)))
    q = _sc_gather_fn()(table, inds)
    out = q[:, :_D].reshape(_B, 32, 32, _D).transpose(0, 3, 1, 2)
    vq_loss = losssum[0, 0] * (1.25 / _N / _D)
    return out, vq_loss
